# baseline (speedup 1.0000x reference)
import functools

import jax
import jax.numpy as jnp
from jax import lax
from jax.experimental import pallas as pl
from jax.experimental.pallas import tpu as pltpu
from jax.experimental.pallas import tpu_sc as plsc

NC = 2
NS = 16
NW = NC * NS
LANES = 16
SLOTS = 126


def _sc_body(C,
             coordw_hbm, field_hbm, compflat_hbm, cflat_hbm, obs_hbm, n_hbm,
             out_hbm, obsidx, cidx, craw, nraw, ctab, cbuf, outbuf, sem):
    wid = lax.axis_index("s") * NC + lax.axis_index("c")
    B = out_hbm.shape[0]
    rows_per = B // NW
    nch = rows_per // C
    iota = lax.broadcasted_iota(jnp.int32, (LANES,), 0)

    pltpu.sync_copy(compflat_hbm, ctab)

    @pl.loop(0, nch)
    def _chunk(g):
        r0 = wid * rows_per + g * C
        pltpu.sync_copy(obs_hbm.at[pl.ds(r0, C)], obsidx)
        pltpu.sync_copy(cflat_hbm.at[pl.ds(2 * r0, 2 * C)],
                        craw.at[pl.ds(LANES, 2 * C)])
        pltpu.sync_copy(n_hbm.at[pl.ds(r0, C)], nraw.at[pl.ds(LANES, C)])
        for k in range((2 * C) // LANES):
            p = iota + (k * LANES)
            c = craw[pl.ds(LANES + k * LANES, LANES)]
            pos = jnp.right_shift(p, 1) * 16 + jnp.bitwise_and(p, 1)
            plsc.store_scatter(cidx, [pos], jnp.right_shift(c, 2))
        cps = []
        for i in range(C):
            cps.append(pltpu.async_copy(
                field_hbm.at[obsidx.at[i]],
                outbuf.at[i, pl.ds(4, 128)], sem))
            cps.append(pltpu.async_copy(
                coordw_hbm.at[cidx.at[pl.ds(16 * i, 2)]],
                cbuf.at[pl.ds(2 * i, 2)], sem))
        for cp in cps:
            cp.wait()
        for i in range(C):
            for j in range(2):
                cj = plsc.load_gather(
                    craw, [jnp.full((LANES,), LANES + 2 * i + j, jnp.int32)])
                sub = jnp.bitwise_and(cj, 3)
                for s2 in range(2):
                    v = jnp.zeros((LANES,), jnp.float32)
                    for q in range(4):
                        w = cbuf[2 * i + j, pl.ds(q * 32 + s2 * 16, LANES)]
                        v = jnp.where(sub == q, w, v)
                    outbuf[i, 2 * j + s2] = v
            nj = plsc.load_gather(
                nraw, [jnp.full((LANES,), LANES + i, jnp.int32)])
            outbuf[i, SLOTS - 1] = plsc.load_gather(ctab, [nj * 16 + iota])
        pltpu.sync_copy(outbuf.at[:, pl.ds(0, SLOTS)],
                        out_hbm.at[pl.ds(r0, C)])


def kernel(coords, obses, n_completed, coord_table, field_table,
           completed_table):
    B = coords.shape[0]
    coords = coords.astype(jnp.int32)
    obses = obses.astype(jnp.int32)
    n_completed = n_completed.astype(jnp.int32)
    fdim = field_table.shape[1]
    coordw = coord_table.reshape(-1, 128)
    obsp = jnp.pad(obses.reshape(B, -1), ((0, 0), (0, 7)))
    compflat = completed_table.reshape(-1)
    cflat = coords.reshape(-1)
    nflat = n_completed.reshape(-1)

    C = 32
    mesh = plsc.VectorSubcoreMesh(core_axis_name="c", subcore_axis_name="s")
    out = pl.kernel(
        functools.partial(_sc_body, C),
        out_type=jax.ShapeDtypeStruct((B, SLOTS, fdim), jnp.float32),
        mesh=mesh,
        compiler_params=pltpu.CompilerParams(
            use_tc_tiling_on_sc=False,
            needs_layout_passes=False,
        ),
        scratch_types=[
            pltpu.VMEM((C, 128), jnp.int32),
            pltpu.VMEM((16 * C,), jnp.int32),
            pltpu.VMEM((2 * C + 16,), jnp.int32),
            pltpu.VMEM((C + 16,), jnp.int32),
            pltpu.VMEM((101 * 16,), jnp.float32),
            pltpu.VMEM((2 * C, 128), jnp.float32),
            pltpu.VMEM((C, SLOTS + 7, fdim), jnp.float32),
            pltpu.SemaphoreType.DMA,
        ],
    )(coordw, field_table, compflat, cflat, obsp, nflat)
    return out.reshape(B, SLOTS * fdim)


# speedup vs baseline: 1.0446x; 1.0446x over previous
import functools

import jax
import jax.numpy as jnp
from jax import lax
from jax.experimental import pallas as pl
from jax.experimental.pallas import tpu as pltpu
from jax.experimental.pallas import tpu_sc as plsc

NC = 2
NS = 16
NW = NC * NS
LANES = 16
SLOTS = 126


def _sc_body(C,
             coord2_hbm, comb_hbm, cflat_hbm, obs_hbm,
             out_hbm, obsidx, cidx, craw, outbuf, sem):
    wid = lax.axis_index("s") * NC + lax.axis_index("c")
    B = out_hbm.shape[0]
    rows_per = B // NW
    nch = rows_per // C
    iota = lax.broadcasted_iota(jnp.int32, (LANES,), 0)

    @pl.loop(0, nch)
    def _chunk(g):
        r0 = wid * rows_per + g * C
        pltpu.sync_copy(obs_hbm.at[pl.ds(r0, C)], obsidx)
        pltpu.sync_copy(cflat_hbm.at[pl.ds(2 * r0, 2 * C)], craw)
        for k in range((2 * C) // LANES):
            p = iota + (k * LANES)
            c = craw[pl.ds(k * LANES, LANES)]
            pos = jnp.right_shift(p, 1) * 16 + jnp.bitwise_and(p, 1) * 2
            plsc.store_scatter(cidx, [pos], c * 2)
            plsc.store_scatter(cidx, [pos + 1], c * 2 + 1)
        cps = []
        for i in range(C):
            cps.append(pltpu.async_copy(
                comb_hbm.at[obsidx.at[i]],
                outbuf.at[i, pl.ds(4, 128)], sem))
            cps.append(pltpu.async_copy(
                coord2_hbm.at[cidx.at[pl.ds(16 * i, 4)]],
                outbuf.at[i, pl.ds(0, 4)], sem))
        for cp in cps:
            cp.wait()
        pltpu.sync_copy(outbuf.at[:, pl.ds(0, SLOTS)],
                        out_hbm.at[pl.ds(r0, C)])


def kernel(coords, obses, n_completed, coord_table, field_table,
           completed_table):
    B = coords.shape[0]
    coords = coords.astype(jnp.int32)
    obses = obses.astype(jnp.int32)
    n_completed = n_completed.astype(jnp.int32)
    fdim = field_table.shape[1]
    nrow_off = field_table.shape[0]
    coord2 = coord_table.reshape(-1, fdim)
    comb = jnp.concatenate([field_table, completed_table], axis=0)

    obsp = jnp.concatenate(
        [obses.reshape(B, -1),
         n_completed.reshape(B, 1) + nrow_off,
         jnp.zeros((B, 6), jnp.int32)], axis=1)
    cflat = coords.reshape(-1)
    obsp, cflat = lax.optimization_barrier((obsp, cflat))

    C = 32
    mesh = plsc.VectorSubcoreMesh(core_axis_name="c", subcore_axis_name="s")
    out = pl.kernel(
        functools.partial(_sc_body, C),
        out_type=jax.ShapeDtypeStruct((B, SLOTS, fdim), jnp.float32),
        mesh=mesh,
        compiler_params=pltpu.CompilerParams(
            use_tc_tiling_on_sc=False,
            needs_layout_passes=False,
        ),
        scratch_types=[
            pltpu.VMEM((C, 128), jnp.int32),
            pltpu.VMEM((16 * C,), jnp.int32),
            pltpu.VMEM((2 * C,), jnp.int32),
            pltpu.VMEM((C, SLOTS + 7, fdim), jnp.float32),
            pltpu.SemaphoreType.DMA,
        ],
    )(coord2, comb, cflat, obsp)
    return out.reshape(B, SLOTS * fdim)
